# baseline (speedup 1.0000x reference)
import jax
import jax.numpy as jnp
from jax import lax
from jax.experimental import pallas as pl
from jax.experimental.pallas import tpu as pltpu
from jax.experimental.pallas import tpu_sc as plsc

_USER_SIZE = 500000
_DIM = 64
_BATCH = 16384
_MARGIN = 1.0

_NC = 2
_NS = 16
_NW = _NC * _NS
_RPW = _BATCH // _NW
_CHUNK = 128
_NCHUNK = _RPW // _CHUNK
_L = 16


def _sc_body(u_hbm, ip_hbm, jn_hbm, it1_hbm, it2_hbm, ent_hbm, rel_hbm,
             out_hbm,
             uidx_v, iidx_v, jidx_v, bcs_v, it1_v, it2_v,
             h0_v, tp0_v, tn0_v, h1_v, tp1_v, tn1_v,
             rel_v, out_v, hsh_v, sem0, sem1):
    wid = lax.axis_index("s") * _NC + lax.axis_index("c")
    base = wid * _RPW

    bufs = ((h0_v, tp0_v, tn0_v, sem0), (h1_v, tp1_v, tn1_v, sem1))

    pltpu.sync_copy(u_hbm.at[pl.ds(base, _RPW)], uidx_v)
    pltpu.sync_copy(ip_hbm.at[pl.ds(base, _RPW)], iidx_v)
    pltpu.sync_copy(jn_hbm.at[pl.ds(base, _RPW)], jidx_v)
    pltpu.sync_copy(it1_hbm.at[pl.ds(base, _RPW)], it1_v)
    pltpu.sync_copy(it2_hbm.at[pl.ds(base, _RPW)], it2_v)

    pltpu.sync_copy(rel_hbm.at[pl.ds(0, 3), :], rel_v.at[pl.ds(0, 3), :])

    def _prep(k, _):
        sl = pl.ds(k * _L, _L)
        iidx_v[sl] = iidx_v[sl] + _USER_SIZE
        jidx_v[sl] = jidx_v[sl] + _USER_SIZE
        bcs_v[sl] = jnp.maximum(jnp.maximum(it1_v[sl], 2 * it2_v[sl]), 0)
        return 0

    lax.fori_loop(0, _RPW // _L, _prep, 0)

    iota = lax.broadcasted_iota(jnp.int32, (_L,), 0)

    sid = lax.axis_index("s")

    def _fire(s, hb, tpb, tnb, sm):
        def _fire_g(g, _):
            off = s * _CHUNK + g * _L
            iu = uidx_v[pl.ds(off, _L)]
            ii = iidx_v[pl.ds(off, _L)]
            ij = jidx_v[pl.ds(off, _L)]
            cs = []
            for k in range(_L):
                cs.append(pltpu.make_async_copy(
                    ent_hbm.at[pl.ds(iu[k], 1), :],
                    hsh_v.at[sid, 0, pl.ds(k, 1), :], sm))
                cs.append(pltpu.make_async_copy(
                    ent_hbm.at[pl.ds(ii[k], 1), :],
                    hsh_v.at[sid, 1, pl.ds(k, 1), :], sm))
                cs.append(pltpu.make_async_copy(
                    ent_hbm.at[pl.ds(ij[k], 1), :],
                    hsh_v.at[sid, 2, pl.ds(k, 1), :], sm))
            for c in cs:
                c.start()
            for c in cs:
                c.wait()
            return 0

        lax.fori_loop(0, _CHUNK // _L, _fire_g, 0)

    def _drain(hb, tpb, tnb, sm):
        pass

    def _compute(s, hb, tpb, tnb):
        def _group(g, _):
            off = s * _CHUNK + g * _L
            bv = bcs_v[pl.ds(off, _L)]
            rows = g * _L + iota
            mbp = jnp.zeros((_L,), jnp.float32)
            mbn = jnp.zeros((_L,), jnp.float32)
            rcp = jnp.zeros((_L,), jnp.float32)
            rcn = jnp.zeros((_L,), jnp.float32)
            for d in range(_DIM):
                cold = jnp.full((_L,), d, jnp.int32)
                h = plsc.load_gather(hb, [rows, cold])
                tp = plsc.load_gather(tpb, [rows, cold])
                tn = plsc.load_gather(tnb, [rows, cold])
                r = plsc.load_gather(rel_v, [bv, cold])
                hr = h + r
                dp = hr - tp
                dn = hr - tn
                ep = h - tp
                en = h - tn
                mbp = mbp + dp * dp
                mbn = mbn + dn * dn
                rcp = rcp + ep * ep
                rcn = rcn + en * en
            out_v[pl.ds(off, _L)] = mbp
            out_v[pl.ds(_RPW + off, _L)] = mbn
            out_v[pl.ds(2 * _RPW + off, _L)] = rcp
            out_v[pl.ds(3 * _RPW + off, _L)] = rcn
            return 0

        lax.fori_loop(0, _CHUNK // _L, _group, 0)

    _fire(0, *bufs[0])
    for s in range(_NCHUNK):
        if s + 1 < _NCHUNK:
            _fire(s + 1, *bufs[(s + 1) % 2])
        hb, tpb, tnb, sm = bufs[s % 2]
        _drain(hb, tpb, tnb, sm)

    for d in range(4):
        pltpu.sync_copy(out_v.at[pl.ds(d * _RPW, _RPW)],
                        out_hbm.at[pl.ds(d * _BATCH + base, _RPW)])


@jax.jit
def _sc_distances(u, ip, jn, it1, it2, ent, rel):
    mesh = plsc.VectorSubcoreMesh(core_axis_name="c", subcore_axis_name="s")
    f = pl.kernel(
        _sc_body,
        mesh=mesh,
        compiler_params=pltpu.CompilerParams(
            needs_layout_passes=False, use_tc_tiling_on_sc=True),
        out_type=jax.ShapeDtypeStruct((4 * _BATCH,), jnp.float32),
        scratch_types=[
            pltpu.VMEM((_RPW,), jnp.int32),
            pltpu.VMEM((_RPW,), jnp.int32),
            pltpu.VMEM((_RPW,), jnp.int32),
            pltpu.VMEM((_RPW,), jnp.int32),
            pltpu.VMEM((_RPW,), jnp.int32),
            pltpu.VMEM((_RPW,), jnp.int32),
            pltpu.VMEM((_CHUNK, _DIM), jnp.float32),
            pltpu.VMEM((_CHUNK, _DIM), jnp.float32),
            pltpu.VMEM((_CHUNK, _DIM), jnp.float32),
            pltpu.VMEM((_CHUNK, _DIM), jnp.float32),
            pltpu.VMEM((_CHUNK, _DIM), jnp.float32),
            pltpu.VMEM((_CHUNK, _DIM), jnp.float32),
            pltpu.VMEM((3, _DIM), jnp.float32),
            pltpu.VMEM((4 * _RPW,), jnp.float32),
            pltpu.VMEM_SHARED((_NS, 3, _L, _DIM), jnp.float32),
            pltpu.SemaphoreType.DMA,
            pltpu.SemaphoreType.DMA,
        ],
    )
    return f(u, ip, jn, it1, it2, ent, rel)


def _tc_body(sq_ref, out_ref):
    tot = jnp.float32(0.0)
    step = 2048
    for j in range(_BATCH // step):
        o = j * step
        a = jnp.sqrt(sq_ref[pl.ds(o, step)])
        b = jnp.sqrt(sq_ref[pl.ds(_BATCH + o, step)])
        c = jnp.sqrt(sq_ref[pl.ds(2 * _BATCH + o, step)])
        d = jnp.sqrt(sq_ref[pl.ds(3 * _BATCH + o, step)])
        tot = tot + jnp.sum(jnp.maximum(a - b + _MARGIN, 0.0))
        tot = tot + jnp.sum(jnp.maximum(c - d + _MARGIN, 0.0))
    out_ref[0, 0] = tot / _BATCH


def kernel(u_batch, i_batch, j_batch, interactions, positive_triples,
           negative_triples, entity_table, relation_table):
    u = u_batch.astype(jnp.int32)
    ip = i_batch.astype(jnp.int32)
    jn = j_batch.astype(jnp.int32)
    it1 = interactions[:, 1].astype(jnp.int32)
    it2 = interactions[:, 2].astype(jnp.int32)
    sq = _sc_distances(u, ip, jn, it1, it2, entity_table, relation_table)
    loss = pl.pallas_call(
        _tc_body,
        out_shape=jax.ShapeDtypeStruct((1, 1), jnp.float32),
        out_specs=pl.BlockSpec(memory_space=pltpu.SMEM),
    )(sq)
    return jnp.reshape(loss, ())


# speedup vs baseline: 1.0671x; 1.0671x over previous
import jax
import jax.numpy as jnp
from jax import lax
from jax.experimental import pallas as pl
from jax.experimental.pallas import tpu as pltpu
from jax.experimental.pallas import tpu_sc as plsc

_USER_SIZE = 500000
_DIM = 64
_BATCH = 16384
_MARGIN = 1.0

_NC = 2
_NS = 16
_NW = _NC * _NS
_RPW = _BATCH // _NW
_CHUNK = 128
_NCHUNK = _RPW // _CHUNK
_L = 16


def _sc_body(u_hbm, ip_hbm, jn_hbm, it1_hbm, it2_hbm, ent_hbm, rel_hbm,
             out_hbm,
             uidx_v, iidx_v, jidx_v, bcs_v, it1_v, it2_v,
             h0_v, tp0_v, tn0_v, h1_v, tp1_v, tn1_v,
             rel_v, out_v, sem0, sem1):
    wid = lax.axis_index("s") * _NC + lax.axis_index("c")
    base = wid * _RPW

    bufs = ((h0_v, tp0_v, tn0_v, sem0), (h1_v, tp1_v, tn1_v, sem1))

    pltpu.sync_copy(u_hbm.at[pl.ds(base, _RPW)], uidx_v)
    pltpu.sync_copy(ip_hbm.at[pl.ds(base, _RPW)], iidx_v)
    pltpu.sync_copy(jn_hbm.at[pl.ds(base, _RPW)], jidx_v)
    pltpu.sync_copy(it1_hbm.at[pl.ds(base, _RPW)], it1_v)
    pltpu.sync_copy(it2_hbm.at[pl.ds(base, _RPW)], it2_v)

    pltpu.sync_copy(rel_hbm.at[pl.ds(0, 3), :], rel_v.at[pl.ds(0, 3), :])

    def _prep(k, _):
        sl = pl.ds(k * _L, _L)
        iidx_v[sl] = iidx_v[sl] + _USER_SIZE
        jidx_v[sl] = jidx_v[sl] + _USER_SIZE
        bcs_v[sl] = jnp.maximum(jnp.maximum(it1_v[sl], 2 * it2_v[sl]), 0)
        return 0

    lax.fori_loop(0, _RPW // _L, _prep, 0)

    iota = lax.broadcasted_iota(jnp.int32, (_L,), 0)

    def _fire(s, hb, tpb, tnb, sm):
        def _fire_g(g, _):
            off = s * _CHUNK + g * _L
            iu = uidx_v[pl.ds(off, _L)]
            ii = iidx_v[pl.ds(off, _L)]
            ij = jidx_v[pl.ds(off, _L)]
            for k in range(_L):
                dst = pl.ds(g * _L + k, 1)
                pltpu.async_copy(ent_hbm.at[pl.ds(iu[k], 1), :],
                                 hb.at[dst, :], sm)
                pltpu.async_copy(ent_hbm.at[pl.ds(ii[k], 1), :],
                                 tpb.at[dst, :], sm)
                pltpu.async_copy(ent_hbm.at[pl.ds(ij[k], 1), :],
                                 tnb.at[dst, :], sm)
            return 0

        lax.fori_loop(0, _CHUNK // _L, _fire_g, 0)

    def _drain(hb, tpb, tnb, sm):
        for buf in (hb, tpb, tnb):
            pltpu.make_async_copy(
                ent_hbm.at[pl.ds(0, _CHUNK), :], buf, sm).wait()

    def _compute(s, hb, tpb, tnb):
        def _group(g, _):
            off = s * _CHUNK + g * _L
            bv = bcs_v[pl.ds(off, _L)]
            rows = g * _L + iota
            mbp = jnp.zeros((_L,), jnp.float32)
            mbn = jnp.zeros((_L,), jnp.float32)
            rcp = jnp.zeros((_L,), jnp.float32)
            rcn = jnp.zeros((_L,), jnp.float32)
            for d in range(_DIM):
                cold = jnp.full((_L,), d, jnp.int32)
                h = plsc.load_gather(hb, [rows, cold])
                tp = plsc.load_gather(tpb, [rows, cold])
                tn = plsc.load_gather(tnb, [rows, cold])
                r = plsc.load_gather(rel_v, [bv, cold])
                hr = h + r
                dp = hr - tp
                dn = hr - tn
                ep = h - tp
                en = h - tn
                mbp = mbp + dp * dp
                mbn = mbn + dn * dn
                rcp = rcp + ep * ep
                rcn = rcn + en * en
            out_v[pl.ds(off, _L)] = mbp
            out_v[pl.ds(_RPW + off, _L)] = mbn
            out_v[pl.ds(2 * _RPW + off, _L)] = rcp
            out_v[pl.ds(3 * _RPW + off, _L)] = rcn
            return 0

        lax.fori_loop(0, _CHUNK // _L, _group, 0)

    _fire(0, *bufs[0])
    for s in range(_NCHUNK):
        if s + 1 < _NCHUNK:
            _fire(s + 1, *bufs[(s + 1) % 2])
        hb, tpb, tnb, sm = bufs[s % 2]
        _drain(hb, tpb, tnb, sm)
        _compute(s, hb, tpb, tnb)

    for d in range(4):
        pltpu.sync_copy(out_v.at[pl.ds(d * _RPW, _RPW)],
                        out_hbm.at[pl.ds(d * _BATCH + base, _RPW)])


@jax.jit
def _sc_distances(u, ip, jn, it1, it2, ent, rel):
    mesh = plsc.VectorSubcoreMesh(core_axis_name="c", subcore_axis_name="s")
    f = pl.kernel(
        _sc_body,
        mesh=mesh,
        compiler_params=pltpu.CompilerParams(
            needs_layout_passes=False, use_tc_tiling_on_sc=True),
        out_type=jax.ShapeDtypeStruct((4 * _BATCH,), jnp.float32),
        scratch_types=[
            pltpu.VMEM((_RPW,), jnp.int32),
            pltpu.VMEM((_RPW,), jnp.int32),
            pltpu.VMEM((_RPW,), jnp.int32),
            pltpu.VMEM((_RPW,), jnp.int32),
            pltpu.VMEM((_RPW,), jnp.int32),
            pltpu.VMEM((_RPW,), jnp.int32),
            pltpu.VMEM((_CHUNK, _DIM), jnp.float32),
            pltpu.VMEM((_CHUNK, _DIM), jnp.float32),
            pltpu.VMEM((_CHUNK, _DIM), jnp.float32),
            pltpu.VMEM((_CHUNK, _DIM), jnp.float32),
            pltpu.VMEM((_CHUNK, _DIM), jnp.float32),
            pltpu.VMEM((_CHUNK, _DIM), jnp.float32),
            pltpu.VMEM((3, _DIM), jnp.float32),
            pltpu.VMEM((4 * _RPW,), jnp.float32),
            pltpu.SemaphoreType.DMA,
            pltpu.SemaphoreType.DMA,
        ],
    )
    return f(u, ip, jn, it1, it2, ent, rel)


def _tc_body(sq_ref, out_ref):
    tot = jnp.float32(0.0)
    step = 2048
    for j in range(_BATCH // step):
        o = j * step
        a = jnp.sqrt(sq_ref[pl.ds(o, step)])
        b = jnp.sqrt(sq_ref[pl.ds(_BATCH + o, step)])
        c = jnp.sqrt(sq_ref[pl.ds(2 * _BATCH + o, step)])
        d = jnp.sqrt(sq_ref[pl.ds(3 * _BATCH + o, step)])
        tot = tot + jnp.sum(jnp.maximum(a - b + _MARGIN, 0.0))
        tot = tot + jnp.sum(jnp.maximum(c - d + _MARGIN, 0.0))
    out_ref[0, 0] = tot / _BATCH


def kernel(u_batch, i_batch, j_batch, interactions, positive_triples,
           negative_triples, entity_table, relation_table):
    u = u_batch.astype(jnp.int32)
    ip = i_batch.astype(jnp.int32)
    jn = j_batch.astype(jnp.int32)
    it1 = interactions[:, 1].astype(jnp.int32)
    it2 = interactions[:, 2].astype(jnp.int32)
    sq = _sc_distances(u, ip, jn, it1, it2, entity_table, relation_table)
    loss = pl.pallas_call(
        _tc_body,
        out_shape=jax.ShapeDtypeStruct((1, 1), jnp.float32),
        out_specs=pl.BlockSpec(memory_space=pltpu.SMEM),
    )(sq)
    return jnp.reshape(loss, ())
